# baseline (speedup 1.0000x reference)
import functools

import jax
import jax.numpy as jnp
from jax.experimental import pallas as pl
from jax.experimental.pallas import tpu as pltpu

_SEQ = 20
_N = 256
_RNN = 128
_G = 4
_G2 = _G * _G
_EMB = 64
_INP = 2
_OUT = 5
_CHUNK = 5


def _body(gt_ref, xy_ref, idr_ref, idc_ref, pc_ref, h0_ref, c0_ref, win_ref,
          bin_ref, wt_ref, bt_ref, wih_ref, bih_ref, whh_ref, bhh_ref,
          wout_ref, bout_ref, out_ref, h_ref, c_ref, *, last_chunk):
    t = pl.program_id(0)

    @pl.when(t == 0)
    def _init():
        h_ref[...] = h0_ref[...]
        c_ref[...] = c0_ref[...]

    idx_row = idr_ref[0]
    idx_col = idc_ref[0]
    p_col = pc_ref[0]
    iota0 = jax.lax.broadcasted_iota(jnp.int32, (_N, _N), 0)
    iota1 = jax.lax.broadcasted_iota(jnp.int32, (_N, _N), 1)
    qp_mat = (p_col == iota1).astype(jnp.float32)
    qx_mat = (idx_col == iota1).astype(jnp.float32)
    qt_mat = (iota0 == idx_row).astype(jnp.float32)

    hi = jax.lax.Precision.HIGHEST
    h_cur = jnp.dot(qp_mat, h_ref[...])
    c_cur = jnp.dot(qp_mat, c_ref[...], precision=hi)
    x_cur = jnp.dot(qx_mat, xy_ref[0])

    s2 = jnp.dot(gt_ref[0].astype(jnp.float32), h_cur)
    social = jnp.concatenate(
        [s2[g * _N:(g + 1) * _N] for g in range(_G2)], axis=1)

    inp_emb = jax.nn.relu(jnp.dot(x_cur, win_ref[...]) + bin_ref[...])
    ten_emb = jax.nn.relu(jnp.dot(social, wt_ref[...]) + bt_ref[...])
    concat = jnp.concatenate([inp_emb, ten_emb], axis=1)

    gates = (jnp.dot(concat, wih_ref[...]) + bih_ref[...]
             + jnp.dot(h_cur, whh_ref[...]) + bhh_ref[...])
    gi = jax.nn.sigmoid(gates[:, :_RNN])
    gf = jax.nn.sigmoid(gates[:, _RNN:2 * _RNN])
    gg = jnp.tanh(gates[:, 2 * _RNN:3 * _RNN])
    go = jax.nn.sigmoid(gates[:, 3 * _RNN:])
    c_new = gf * c_cur + gi * gg
    h_new = go * jnp.tanh(c_new)
    out_t = jnp.dot(h_new, wout_ref[...]) + bout_ref[...]

    out_ref[0] = jnp.dot(qt_mat, out_t, precision=hi)

    if last_chunk:
        @pl.when(t < _CHUNK - 1)
        def _carry():
            h_ref[...] = h_new
            c_ref[...] = c_new

        @pl.when(t == _CHUNK - 1)
        def _final():
            h_ref[...] = jnp.dot(qt_mat, h_new, precision=hi)
            c_ref[...] = jnp.dot(qt_mat, c_new, precision=hi)
    else:
        h_ref[...] = h_new
        c_ref[...] = c_new


def kernel(pedxy, hidden_states, cell_states, outputs, grids, node_ids,
           W_in, b_in, W_t, b_t, W_ih, b_ih, W_hh, b_hh, W_out, b_out):
    del outputs

    idr = node_ids.reshape(_SEQ, 1, _N)
    inv = jnp.argsort(node_ids, axis=1).astype(jnp.int32)
    p = jnp.concatenate(
        [node_ids[:1], jnp.take_along_axis(inv[:-1], node_ids[1:], axis=1)],
        axis=0)
    pc = p.reshape(_SEQ, _N, 1)
    idc = node_ids.reshape(_SEQ, _N, 1)

    full = lambda shape: pl.BlockSpec(shape, lambda t: (0,) * len(shape))
    per_t = lambda shape: pl.BlockSpec(shape, lambda t: (t,) + (0,) * (len(shape) - 1))

    n_chunks = _SEQ // _CHUNK
    h_st, c_st = hidden_states, cell_states
    outs = []
    for k in range(n_chunks):
        lo = k * _CHUNK
        gt_k = grids[lo:lo + _CHUNK].astype(jnp.bfloat16).transpose(
            0, 3, 1, 2).reshape(_CHUNK, _G2 * _N, _N)
        out_k, h_st, c_st = pl.pallas_call(
            functools.partial(_body, last_chunk=(k == n_chunks - 1)),
            grid=(_CHUNK,),
            in_specs=[
                per_t((1, _G2 * _N, _N)),
                per_t((1, _N, _INP)),
                per_t((1, 1, _N)),
                per_t((1, _N, 1)),
                per_t((1, _N, 1)),
                full((_N, _RNN)),
                full((_N, _RNN)),
                full((_INP, _EMB)),
                full((1, _EMB)),
                full((_G2 * _RNN, _EMB)),
                full((1, _EMB)),
                full((2 * _EMB, 4 * _RNN)),
                full((1, 4 * _RNN)),
                full((_RNN, 4 * _RNN)),
                full((1, 4 * _RNN)),
                full((_RNN, _OUT)),
                full((1, _OUT)),
            ],
            out_specs=(
                per_t((1, _N, _OUT)),
                full((_N, _RNN)),
                full((_N, _RNN)),
            ),
            out_shape=(
                jax.ShapeDtypeStruct((_CHUNK, _N, _OUT), jnp.float32),
                jax.ShapeDtypeStruct((_N, _RNN), jnp.float32),
                jax.ShapeDtypeStruct((_N, _RNN), jnp.float32),
            ),
            compiler_params=pltpu.CompilerParams(
                dimension_semantics=("arbitrary",)),
        )(gt_k, pedxy[lo:lo + _CHUNK], idr[lo:lo + _CHUNK],
          idc[lo:lo + _CHUNK], pc[lo:lo + _CHUNK],
          h_st, c_st, W_in, b_in.reshape(1, _EMB), W_t, b_t.reshape(1, _EMB),
          W_ih.T, b_ih.reshape(1, 4 * _RNN), W_hh.T,
          b_hh.reshape(1, 4 * _RNN), W_out.T, b_out.reshape(1, _OUT))
        outs.append(out_k)

    return jnp.concatenate(outs, axis=0), h_st, c_st


# speedup vs baseline: 1.3778x; 1.3778x over previous
import jax
import jax.numpy as jnp
from jax.experimental import pallas as pl
from jax.experimental.pallas import tpu as pltpu

_SEQ = 20
_N = 256
_RNN = 128
_G = 4
_G2 = _G * _G
_EMB = 64
_INP = 2
_OUT = 5


def _body(gt_ref, xy_ref, idr_ref, idc_ref, pc_ref, h0_ref, c0_ref, win_ref,
          bin_ref, wt_ref, bt_ref, wih_ref, bih_ref, whh_ref, bhh_ref,
          wout_ref, bout_ref, out_ref, h_ref, c_ref):
    t = pl.program_id(0)

    @pl.when(t == 0)
    def _init():
        h_ref[...] = h0_ref[...]
        c_ref[...] = c0_ref[...]

    idx_row = idr_ref[0]
    idx_col = idc_ref[0]
    p_col = pc_ref[0]
    iota0 = jax.lax.broadcasted_iota(jnp.int32, (_N, _N), 0)
    iota1 = jax.lax.broadcasted_iota(jnp.int32, (_N, _N), 1)
    qp_mat = (p_col == iota1).astype(jnp.float32)
    qx_mat = (idx_col == iota1).astype(jnp.float32)
    qt_mat = (iota0 == idx_row).astype(jnp.float32)

    hi = jax.lax.Precision.HIGHEST
    h_cur = jnp.dot(qp_mat, h_ref[...])
    c_cur = jnp.dot(qp_mat, c_ref[...], precision=hi)
    x_cur = jnp.dot(qx_mat, xy_ref[0])

    s2 = jnp.dot(gt_ref[0].astype(jnp.float32), h_cur)
    social = jnp.concatenate(
        [s2[g * _N:(g + 1) * _N] for g in range(_G2)], axis=1)

    inp_emb = jax.nn.relu(jnp.dot(x_cur, win_ref[...]) + bin_ref[...])
    ten_emb = jax.nn.relu(jnp.dot(social, wt_ref[...]) + bt_ref[...])
    concat = jnp.concatenate([inp_emb, ten_emb], axis=1)

    gates = (jnp.dot(concat, wih_ref[...]) + bih_ref[...]
             + jnp.dot(h_cur, whh_ref[...]) + bhh_ref[...])
    gi = jax.nn.sigmoid(gates[:, :_RNN])
    gf = jax.nn.sigmoid(gates[:, _RNN:2 * _RNN])
    gg = jnp.tanh(gates[:, 2 * _RNN:3 * _RNN])
    go = jax.nn.sigmoid(gates[:, 3 * _RNN:])
    c_new = gf * c_cur + gi * gg
    h_new = go * jnp.tanh(c_new)
    out_t = jnp.dot(h_new, wout_ref[...]) + bout_ref[...]

    out_ref[0] = jnp.dot(qt_mat, out_t, precision=hi)

    @pl.when(t < _SEQ - 1)
    def _carry():
        h_ref[...] = h_new
        c_ref[...] = c_new

    @pl.when(t == _SEQ - 1)
    def _final():
        h_ref[...] = jnp.dot(qt_mat, h_new, precision=hi)
        c_ref[...] = jnp.dot(qt_mat, c_new, precision=hi)


def kernel(pedxy, hidden_states, cell_states, outputs, grids, node_ids,
           W_in, b_in, W_t, b_t, W_ih, b_ih, W_hh, b_hh, W_out, b_out):
    del outputs

    gt = grids.astype(jnp.bfloat16).transpose(0, 3, 1, 2).reshape(
        _SEQ, _G2 * _N, _N)
    idr = node_ids.reshape(_SEQ, 1, _N)
    idc = node_ids.reshape(_SEQ, _N, 1)
    inv = jnp.argsort(node_ids, axis=1).astype(jnp.int32)
    p = jnp.concatenate(
        [node_ids[:1], jnp.take_along_axis(inv[:-1], node_ids[1:], axis=1)],
        axis=0)
    pc = p.reshape(_SEQ, _N, 1)

    full = lambda shape: pl.BlockSpec(shape, lambda t: (0,) * len(shape))
    per_t = lambda shape: pl.BlockSpec(shape, lambda t: (t,) + (0,) * (len(shape) - 1))

    outputs_r, h_out, c_out = pl.pallas_call(
        _body,
        grid=(_SEQ,),
        in_specs=[
            per_t((1, _G2 * _N, _N)),
            per_t((1, _N, _INP)),
            per_t((1, 1, _N)),
            per_t((1, _N, 1)),
            per_t((1, _N, 1)),
            full((_N, _RNN)),
            full((_N, _RNN)),
            full((_INP, _EMB)),
            full((1, _EMB)),
            full((_G2 * _RNN, _EMB)),
            full((1, _EMB)),
            full((2 * _EMB, 4 * _RNN)),
            full((1, 4 * _RNN)),
            full((_RNN, 4 * _RNN)),
            full((1, 4 * _RNN)),
            full((_RNN, _OUT)),
            full((1, _OUT)),
        ],
        out_specs=(
            per_t((1, _N, _OUT)),
            full((_N, _RNN)),
            full((_N, _RNN)),
        ),
        out_shape=(
            jax.ShapeDtypeStruct((_SEQ, _N, _OUT), jnp.float32),
            jax.ShapeDtypeStruct((_N, _RNN), jnp.float32),
            jax.ShapeDtypeStruct((_N, _RNN), jnp.float32),
        ),
        compiler_params=pltpu.CompilerParams(
            dimension_semantics=("arbitrary",)),
    )(gt, pedxy, idr, idc, pc, hidden_states, cell_states,
      W_in, b_in.reshape(1, _EMB), W_t, b_t.reshape(1, _EMB),
      W_ih.T, b_ih.reshape(1, 4 * _RNN), W_hh.T, b_hh.reshape(1, 4 * _RNN),
      W_out.T, b_out.reshape(1, _OUT))

    return outputs_r, h_out, c_out
